# baseline (speedup 1.0000x reference)
import functools

import jax
import jax.numpy as jnp
from jax import lax
from jax.experimental import pallas as pl
from jax.experimental.pallas import tpu as pltpu
from jax.experimental.pallas import tpu_sc as plsc

REQ_VEC = 400
LATENT = 64
BATCH = 1024

ITEM_BLOCK = 2048
SUBLANES = 8


def _make_sc_gather(num_users):
    info = plsc.get_sparse_core_info()
    nc = info.num_cores
    b_per_c = BATCH // nc
    mesh = plsc.ScalarSubcoreMesh(axis_name="c", num_cores=nc)

    @functools.partial(
        pl.kernel,
        mesh=mesh,
        out_type=jax.ShapeDtypeStruct((BATCH, REQ_VEC), jnp.float32),
        scratch_types=[
            pltpu.SMEM((b_per_c,), jnp.int32),
            pltpu.SemaphoreType.DMA,
        ],
    )
    def gather_kernel(table_hbm, idx_hbm, out_hbm, idx_s, sem):
        base = lax.axis_index("c") * b_per_c
        pltpu.sync_copy(idx_hbm.at[pl.ds(base, b_per_c)], idx_s)

        def issue(i, _):
            pltpu.make_async_copy(
                table_hbm.at[idx_s[i]], out_hbm.at[base + i], sem
            ).start()
            return ()

        def drain(i, _):
            pltpu.make_async_copy(
                table_hbm.at[idx_s[i]], out_hbm.at[base + i], sem
            ).wait()
            return ()

        lax.fori_loop(0, b_per_c, issue, ())
        lax.fori_loop(0, b_per_c, drain, ())

    return gather_kernel


def _score_body(ug_ref, item_ref, fs_ref, out_ref, fu_ref):
    @pl.when(pl.program_id(0) == 0)
    def _():
        fu_ref[...] = jnp.dot(
            ug_ref[...], fs_ref[...], preferred_element_type=jnp.float32
        )

    fi = jnp.dot(item_ref[...], fs_ref[...], preferred_element_type=jnp.float32)
    logits = lax.dot_general(
        fu_ref[...], fi, (((1,), (1,)), ((), ())),
        preferred_element_type=jnp.float32,
    )
    out_ref[...] = jax.nn.sigmoid(logits)


def _tc_score(ug, item_vector, FS, interpret=False):
    n_items = item_vector.shape[0]
    grid = (pl.cdiv(n_items, ITEM_BLOCK),)
    return pl.pallas_call(
        _score_body,
        grid=grid,
        in_specs=[
            pl.BlockSpec((BATCH, REQ_VEC), lambda i: (0, 0)),
            pl.BlockSpec((ITEM_BLOCK, REQ_VEC), lambda i: (i, 0)),
            pl.BlockSpec((REQ_VEC, LATENT), lambda i: (0, 0)),
        ],
        out_specs=pl.BlockSpec((BATCH, ITEM_BLOCK), lambda i: (0, i)),
        out_shape=jax.ShapeDtypeStruct((BATCH, n_items), jnp.float32),
        scratch_shapes=[pltpu.VMEM((BATCH, LATENT), jnp.float32)],
        interpret=interpret,
    )(ug, item_vector, FS)


@jax.jit
def kernel(users, user_vector, item_vector, FS):
    users = users.astype(jnp.int32)
    ug = jnp.take(user_vector, users, axis=0)
    return _tc_score(ug, item_vector, FS)


# speedup vs baseline: 5.0536x; 5.0536x over previous
import functools

import jax
import jax.numpy as jnp
from jax import lax
from jax.experimental import pallas as pl
from jax.experimental.pallas import tpu as pltpu
from jax.experimental.pallas import tpu_sc as plsc

REQ_VEC = 400
LATENT = 64
LATENT_PAD = 128
BATCH = 1024

USER_BLOCK = 4096
ITEM_BLOCK = 2048


def _au_body(uT_ref, fsp_ref, au_ref):
    au_ref[...] = lax.dot_general(
        uT_ref[...], fsp_ref[...], (((0,), (0,)), ((), ())),
        preferred_element_type=jnp.float32,
    )


def _tc_all_users(uT, FSP, interpret=False):
    n_users = uT.shape[1]
    grid = (pl.cdiv(n_users, USER_BLOCK),)
    return pl.pallas_call(
        _au_body,
        grid=grid,
        in_specs=[
            pl.BlockSpec((REQ_VEC, USER_BLOCK), lambda i: (0, i)),
            pl.BlockSpec((REQ_VEC, LATENT_PAD), lambda i: (0, 0)),
        ],
        out_specs=pl.BlockSpec((USER_BLOCK, LATENT_PAD), lambda i: (i, 0)),
        out_shape=jax.ShapeDtypeStruct((n_users, LATENT_PAD), jnp.float32),
        interpret=interpret,
    )(uT, FSP)


def _make_sc_gather(num_users):
    info = plsc.get_sparse_core_info()
    nw = info.num_cores * info.num_subcores
    assert BATCH % (8 * nw) == 0
    b_per_w = BATCH // nw
    mesh = plsc.VectorSubcoreMesh(core_axis_name="c", subcore_axis_name="s")

    @functools.partial(
        pl.kernel,
        mesh=mesh,
        out_type=jax.ShapeDtypeStruct((BATCH, LATENT_PAD), jnp.float32),
        scratch_types=[
            pltpu.VMEM((b_per_w,), jnp.int32),
            pltpu.VMEM((b_per_w, LATENT_PAD), jnp.float32),
            pltpu.SemaphoreType.DMA,
        ],
    )
    def gather_kernel(table_hbm, idx_hbm, out_hbm, idx_v, rows_v, sem):
        wid = lax.axis_index("s") * info.num_cores + lax.axis_index("c")
        base = wid * b_per_w
        pltpu.sync_copy(idx_hbm.at[pl.ds(base, b_per_w)], idx_v)
        pltpu.async_copy(table_hbm.at[idx_v], rows_v, sem).wait()
        pltpu.sync_copy(rows_v, out_hbm.at[pl.ds(base, b_per_w)])

    return gather_kernel


def _score_body(fst_ref, ug_ref, itemT_ref, out_ref):
    fiT = lax.dot_general(
        fst_ref[...], itemT_ref[...], (((1,), (0,)), ((), ())),
        preferred_element_type=jnp.float32,
    )
    logitsT = lax.dot_general(
        fiT, ug_ref[:, :LATENT], (((0,), (1,)), ((), ())),
        preferred_element_type=jnp.float32,
    )
    out_ref[...] = jax.nn.sigmoid(logitsT)


def _tc_score(FST, ug, itemT, interpret=False):
    n_items = itemT.shape[1]
    grid = (pl.cdiv(n_items, ITEM_BLOCK),)
    return pl.pallas_call(
        _score_body,
        grid=grid,
        in_specs=[
            pl.BlockSpec((LATENT, REQ_VEC), lambda i: (0, 0)),
            pl.BlockSpec((BATCH, LATENT_PAD), lambda i: (0, 0)),
            pl.BlockSpec((REQ_VEC, ITEM_BLOCK), lambda i: (0, i)),
        ],
        out_specs=pl.BlockSpec((ITEM_BLOCK, BATCH), lambda i: (i, 0)),
        out_shape=jax.ShapeDtypeStruct((n_items, BATCH), jnp.float32),
        interpret=interpret,
    )(FST, ug, itemT)


@jax.jit
def kernel(users, user_vector, item_vector, FS):
    users = users.astype(jnp.int32)
    FSP = jnp.pad(FS, ((0, 0), (0, LATENT_PAD - LATENT)))
    au = _tc_all_users(user_vector.T, FSP)
    gather = _make_sc_gather(user_vector.shape[0])
    ug = gather(au, users)
    ratingT = _tc_score(FS.T, ug, item_vector.T)
    return ratingT.T


# speedup vs baseline: 5.1931x; 1.0276x over previous
import functools

import jax
import jax.numpy as jnp
from jax import lax
from jax.experimental import pallas as pl
from jax.experimental.pallas import tpu as pltpu
from jax.experimental.pallas import tpu_sc as plsc

REQ_VEC = 400
LATENT = 64
LATENT_PAD = 128
BATCH = 1024

USER_BLOCK = 4096
ITEM_BLOCK = 4096


def _au_body(uT_ref, fsp_ref, au_ref):
    au_ref[...] = lax.dot_general(
        uT_ref[...], fsp_ref[...], (((0,), (0,)), ((), ())),
        preferred_element_type=jnp.float32,
    )


def _tc_all_users(uT, FSP, interpret=False):
    n_users = uT.shape[1]
    grid = (pl.cdiv(n_users, USER_BLOCK),)
    return pl.pallas_call(
        _au_body,
        grid=grid,
        in_specs=[
            pl.BlockSpec((REQ_VEC, USER_BLOCK), lambda i: (0, i)),
            pl.BlockSpec((REQ_VEC, LATENT_PAD), lambda i: (0, 0)),
        ],
        out_specs=pl.BlockSpec((USER_BLOCK, LATENT_PAD), lambda i: (i, 0)),
        out_shape=jax.ShapeDtypeStruct((n_users, LATENT_PAD), jnp.float32),
        interpret=interpret,
    )(uT, FSP)


def _make_sc_gather(num_users):
    info = plsc.get_sparse_core_info()
    nw = info.num_cores * info.num_subcores
    assert BATCH % (8 * nw) == 0
    b_per_w = BATCH // nw
    mesh = plsc.VectorSubcoreMesh(core_axis_name="c", subcore_axis_name="s")

    @functools.partial(
        pl.kernel,
        mesh=mesh,
        out_type=jax.ShapeDtypeStruct((BATCH, LATENT_PAD), jnp.float32),
        scratch_types=[
            pltpu.VMEM((b_per_w,), jnp.int32),
            pltpu.VMEM((b_per_w, LATENT_PAD), jnp.float32),
            pltpu.SemaphoreType.DMA,
        ],
    )
    def gather_kernel(table_hbm, idx_hbm, out_hbm, idx_v, rows_v, sem):
        wid = lax.axis_index("s") * info.num_cores + lax.axis_index("c")
        base = wid * b_per_w
        pltpu.sync_copy(idx_hbm.at[pl.ds(base, b_per_w)], idx_v)
        pltpu.async_copy(table_hbm.at[idx_v], rows_v, sem).wait()
        pltpu.sync_copy(rows_v, out_hbm.at[pl.ds(base, b_per_w)])

    return gather_kernel


def _score_body(fst_ref, ug_ref, itemT_ref, out_ref):
    fiT = lax.dot_general(
        fst_ref[...], itemT_ref[...], (((1,), (0,)), ((), ())),
        preferred_element_type=jnp.float32,
    )
    logitsT = lax.dot_general(
        fiT, ug_ref[:, :LATENT], (((0,), (1,)), ((), ())),
        preferred_element_type=jnp.float32,
    )
    out_ref[...] = jax.nn.sigmoid(logitsT)


def _tc_score(FST, ug, itemT, interpret=False):
    n_items = itemT.shape[1]
    grid = (pl.cdiv(n_items, ITEM_BLOCK),)
    return pl.pallas_call(
        _score_body,
        grid=grid,
        in_specs=[
            pl.BlockSpec((LATENT, REQ_VEC), lambda i: (0, 0)),
            pl.BlockSpec((BATCH, LATENT_PAD), lambda i: (0, 0)),
            pl.BlockSpec((REQ_VEC, ITEM_BLOCK), lambda i: (0, i)),
        ],
        out_specs=pl.BlockSpec((ITEM_BLOCK, BATCH), lambda i: (i, 0)),
        out_shape=jax.ShapeDtypeStruct((n_items, BATCH), jnp.float32),
        interpret=interpret,
    )(FST, ug, itemT)


@jax.jit
def kernel(users, user_vector, item_vector, FS):
    users = users.astype(jnp.int32)
    FSP = jnp.pad(FS, ((0, 0), (0, LATENT_PAD - LATENT)))
    au = _tc_all_users(user_vector.T, FSP)
    gather = _make_sc_gather(user_vector.shape[0])
    ug = gather(au, users)
    ratingT = _tc_score(FS.T, ug, item_vector.T)
    return ratingT.T


# speedup vs baseline: 5.2179x; 1.0048x over previous
import functools

import jax
import jax.numpy as jnp
from jax import lax
from jax.experimental import pallas as pl
from jax.experimental.pallas import tpu as pltpu
from jax.experimental.pallas import tpu_sc as plsc

REQ_VEC = 400
LATENT = 64
LATENT_PAD = 128
BATCH = 1024

USER_BLOCK = 8192
ITEM_BLOCK = 4096


def _au_body(uT_ref, fsp_ref, au_ref):
    au_ref[...] = lax.dot_general(
        uT_ref[...], fsp_ref[...], (((0,), (0,)), ((), ())),
        preferred_element_type=jnp.float32,
    )


def _tc_all_users(uT, FSP, interpret=False):
    n_users = uT.shape[1]
    grid = (pl.cdiv(n_users, USER_BLOCK),)
    return pl.pallas_call(
        _au_body,
        grid=grid,
        in_specs=[
            pl.BlockSpec((REQ_VEC, USER_BLOCK), lambda i: (0, i)),
            pl.BlockSpec((REQ_VEC, LATENT_PAD), lambda i: (0, 0)),
        ],
        out_specs=pl.BlockSpec((USER_BLOCK, LATENT_PAD), lambda i: (i, 0)),
        out_shape=jax.ShapeDtypeStruct((n_users, LATENT_PAD), jnp.float32),
        interpret=interpret,
    )(uT, FSP)


def _make_sc_gather(num_users):
    info = plsc.get_sparse_core_info()
    nw = info.num_cores * info.num_subcores
    assert BATCH % (8 * nw) == 0
    b_per_w = BATCH // nw
    mesh = plsc.VectorSubcoreMesh(core_axis_name="c", subcore_axis_name="s")

    @functools.partial(
        pl.kernel,
        mesh=mesh,
        out_type=jax.ShapeDtypeStruct((BATCH, LATENT_PAD), jnp.float32),
        scratch_types=[
            pltpu.VMEM((b_per_w,), jnp.int32),
            pltpu.VMEM((b_per_w, LATENT_PAD), jnp.float32),
            pltpu.SemaphoreType.DMA,
        ],
    )
    def gather_kernel(table_hbm, idx_hbm, out_hbm, idx_v, rows_v, sem):
        wid = lax.axis_index("s") * info.num_cores + lax.axis_index("c")
        base = wid * b_per_w
        pltpu.sync_copy(idx_hbm.at[pl.ds(base, b_per_w)], idx_v)
        pltpu.async_copy(table_hbm.at[idx_v], rows_v, sem).wait()
        pltpu.sync_copy(rows_v, out_hbm.at[pl.ds(base, b_per_w)])

    return gather_kernel


def _score_body(fst_ref, ug_ref, itemT_ref, out_ref):
    fiT = lax.dot_general(
        fst_ref[...], itemT_ref[...], (((1,), (0,)), ((), ())),
        preferred_element_type=jnp.float32,
    )
    logitsT = lax.dot_general(
        fiT, ug_ref[:, :LATENT], (((0,), (1,)), ((), ())),
        preferred_element_type=jnp.float32,
    )
    out_ref[...] = jax.nn.sigmoid(logitsT)


def _tc_score(FST, ug, itemT, interpret=False):
    n_items = itemT.shape[1]
    grid = (pl.cdiv(n_items, ITEM_BLOCK),)
    return pl.pallas_call(
        _score_body,
        grid=grid,
        in_specs=[
            pl.BlockSpec((LATENT, REQ_VEC), lambda i: (0, 0)),
            pl.BlockSpec((BATCH, LATENT_PAD), lambda i: (0, 0)),
            pl.BlockSpec((REQ_VEC, ITEM_BLOCK), lambda i: (0, i)),
        ],
        out_specs=pl.BlockSpec((ITEM_BLOCK, BATCH), lambda i: (i, 0)),
        out_shape=jax.ShapeDtypeStruct((n_items, BATCH), jnp.float32),
        interpret=interpret,
    )(FST, ug, itemT)


@jax.jit
def kernel(users, user_vector, item_vector, FS):
    users = users.astype(jnp.int32)
    FSP = jnp.pad(FS, ((0, 0), (0, LATENT_PAD - LATENT)))
    au = _tc_all_users(user_vector.T, FSP)
    gather = _make_sc_gather(user_vector.shape[0])
    ug = gather(au, users)
    ratingT = _tc_score(FS.T, ug, item_vector.T)
    return ratingT.T


# speedup vs baseline: 5.2569x; 1.0075x over previous
import functools

import jax
import jax.numpy as jnp
from jax import lax
from jax.experimental import pallas as pl
from jax.experimental.pallas import tpu as pltpu
from jax.experimental.pallas import tpu_sc as plsc

REQ_VEC = 400
LATENT = 64
LATENT_PAD = 128
BATCH = 1024

USER_BLOCK = 12800
ITEM_BLOCK = 4096


def _au_body(uT_ref, fsp_ref, au_ref):
    au_ref[...] = lax.dot_general(
        uT_ref[...], fsp_ref[...], (((0,), (0,)), ((), ())),
        preferred_element_type=jnp.float32,
    )


def _tc_all_users(uT, FSP, interpret=False):
    n_users = uT.shape[1]
    grid = (pl.cdiv(n_users, USER_BLOCK),)
    return pl.pallas_call(
        _au_body,
        grid=grid,
        in_specs=[
            pl.BlockSpec((REQ_VEC, USER_BLOCK), lambda i: (0, i)),
            pl.BlockSpec((REQ_VEC, LATENT_PAD), lambda i: (0, 0)),
        ],
        out_specs=pl.BlockSpec((USER_BLOCK, LATENT_PAD), lambda i: (i, 0)),
        out_shape=jax.ShapeDtypeStruct((n_users, LATENT_PAD), jnp.float32),
        interpret=interpret,
    )(uT, FSP)


def _make_sc_gather(num_users):
    info = plsc.get_sparse_core_info()
    nw = info.num_cores * info.num_subcores
    assert BATCH % (8 * nw) == 0
    b_per_w = BATCH // nw
    mesh = plsc.VectorSubcoreMesh(core_axis_name="c", subcore_axis_name="s")

    @functools.partial(
        pl.kernel,
        mesh=mesh,
        out_type=jax.ShapeDtypeStruct((BATCH, LATENT_PAD), jnp.float32),
        scratch_types=[
            pltpu.VMEM((b_per_w,), jnp.int32),
            pltpu.VMEM((b_per_w, LATENT_PAD), jnp.float32),
            pltpu.SemaphoreType.DMA,
        ],
    )
    def gather_kernel(table_hbm, idx_hbm, out_hbm, idx_v, rows_v, sem):
        wid = lax.axis_index("s") * info.num_cores + lax.axis_index("c")
        base = wid * b_per_w
        pltpu.sync_copy(idx_hbm.at[pl.ds(base, b_per_w)], idx_v)
        pltpu.async_copy(table_hbm.at[idx_v], rows_v, sem).wait()
        pltpu.sync_copy(rows_v, out_hbm.at[pl.ds(base, b_per_w)])

    return gather_kernel


def _score_body(fst_ref, ug_ref, itemT_ref, out_ref):
    fiT = lax.dot_general(
        fst_ref[...], itemT_ref[...], (((1,), (0,)), ((), ())),
        preferred_element_type=jnp.float32,
    )
    logitsT = lax.dot_general(
        fiT, ug_ref[:, :LATENT], (((0,), (1,)), ((), ())),
        preferred_element_type=jnp.float32,
    )
    out_ref[...] = jax.nn.sigmoid(logitsT)


def _tc_score(FST, ug, itemT, interpret=False):
    n_items = itemT.shape[1]
    grid = (pl.cdiv(n_items, ITEM_BLOCK),)
    return pl.pallas_call(
        _score_body,
        grid=grid,
        in_specs=[
            pl.BlockSpec((LATENT, REQ_VEC), lambda i: (0, 0)),
            pl.BlockSpec((BATCH, LATENT_PAD), lambda i: (0, 0)),
            pl.BlockSpec((REQ_VEC, ITEM_BLOCK), lambda i: (0, i)),
        ],
        out_specs=pl.BlockSpec((ITEM_BLOCK, BATCH), lambda i: (i, 0)),
        out_shape=jax.ShapeDtypeStruct((n_items, BATCH), jnp.float32),
        interpret=interpret,
    )(FST, ug, itemT)


@jax.jit
def kernel(users, user_vector, item_vector, FS):
    users = users.astype(jnp.int32)
    FSP = jnp.pad(FS, ((0, 0), (0, LATENT_PAD - LATENT)))
    au = _tc_all_users(user_vector.T, FSP)
    gather = _make_sc_gather(user_vector.shape[0])
    ug = gather(au, users)
    ratingT = _tc_score(FS.T, ug, item_vector.T)
    return ratingT.T


# speedup vs baseline: 5.2698x; 1.0024x over previous
import functools

import jax
import jax.numpy as jnp
from jax import lax
from jax.experimental import pallas as pl
from jax.experimental.pallas import tpu as pltpu
from jax.experimental.pallas import tpu_sc as plsc

REQ_VEC = 400
LATENT = 64
LATENT_PAD = 128
BATCH = 1024

USER_BLOCK = 12800
ITEM_BLOCK = 5120


def _au_body(uT_ref, fsp_ref, au_ref):
    au_ref[...] = lax.dot_general(
        uT_ref[...], fsp_ref[...], (((0,), (0,)), ((), ())),
        preferred_element_type=jnp.float32,
    )


def _tc_all_users(uT, FSP, interpret=False):
    n_users = uT.shape[1]
    grid = (pl.cdiv(n_users, USER_BLOCK),)
    return pl.pallas_call(
        _au_body,
        grid=grid,
        in_specs=[
            pl.BlockSpec((REQ_VEC, USER_BLOCK), lambda i: (0, i)),
            pl.BlockSpec((REQ_VEC, LATENT_PAD), lambda i: (0, 0)),
        ],
        out_specs=pl.BlockSpec((USER_BLOCK, LATENT_PAD), lambda i: (i, 0)),
        out_shape=jax.ShapeDtypeStruct((n_users, LATENT_PAD), jnp.float32),
        compiler_params=pltpu.CompilerParams(vmem_limit_bytes=66060288),
        interpret=interpret,
    )(uT, FSP)


def _make_sc_gather(num_users):
    info = plsc.get_sparse_core_info()
    nw = info.num_cores * info.num_subcores
    assert BATCH % (8 * nw) == 0
    b_per_w = BATCH // nw
    mesh = plsc.VectorSubcoreMesh(core_axis_name="c", subcore_axis_name="s")

    @functools.partial(
        pl.kernel,
        mesh=mesh,
        out_type=jax.ShapeDtypeStruct((BATCH, LATENT_PAD), jnp.float32),
        scratch_types=[
            pltpu.VMEM((b_per_w,), jnp.int32),
            pltpu.VMEM((b_per_w, LATENT_PAD), jnp.float32),
            pltpu.SemaphoreType.DMA,
        ],
    )
    def gather_kernel(table_hbm, idx_hbm, out_hbm, idx_v, rows_v, sem):
        wid = lax.axis_index("s") * info.num_cores + lax.axis_index("c")
        base = wid * b_per_w
        pltpu.sync_copy(idx_hbm.at[pl.ds(base, b_per_w)], idx_v)
        pltpu.async_copy(table_hbm.at[idx_v], rows_v, sem).wait()
        pltpu.sync_copy(rows_v, out_hbm.at[pl.ds(base, b_per_w)])

    return gather_kernel


def _score_body(fst_ref, ug_ref, itemT_ref, out_ref):
    fiT = lax.dot_general(
        fst_ref[...], itemT_ref[...], (((1,), (0,)), ((), ())),
        preferred_element_type=jnp.float32,
    )
    logitsT = lax.dot_general(
        fiT, ug_ref[:, :LATENT], (((0,), (1,)), ((), ())),
        preferred_element_type=jnp.float32,
    )
    out_ref[...] = jax.nn.sigmoid(logitsT)


def _tc_score(FST, ug, itemT, interpret=False):
    n_items = itemT.shape[1]
    grid = (pl.cdiv(n_items, ITEM_BLOCK),)
    return pl.pallas_call(
        _score_body,
        grid=grid,
        in_specs=[
            pl.BlockSpec((LATENT, REQ_VEC), lambda i: (0, 0)),
            pl.BlockSpec((BATCH, LATENT_PAD), lambda i: (0, 0)),
            pl.BlockSpec((REQ_VEC, ITEM_BLOCK), lambda i: (0, i)),
        ],
        out_specs=pl.BlockSpec((ITEM_BLOCK, BATCH), lambda i: (i, 0)),
        out_shape=jax.ShapeDtypeStruct((n_items, BATCH), jnp.float32),
        compiler_params=pltpu.CompilerParams(vmem_limit_bytes=66060288),
        interpret=interpret,
    )(FST, ug, itemT)


@jax.jit
def kernel(users, user_vector, item_vector, FS):
    users = users.astype(jnp.int32)
    FSP = jnp.pad(FS, ((0, 0), (0, LATENT_PAD - LATENT)))
    au = _tc_all_users(user_vector.T, FSP)
    gather = _make_sc_gather(user_vector.shape[0])
    ug = gather(au, users)
    ratingT = _tc_score(FS.T, ug, item_vector.T)
    return ratingT.T


# speedup vs baseline: 5.2750x; 1.0010x over previous
import functools

import jax
import jax.numpy as jnp
from jax import lax
from jax.experimental import pallas as pl
from jax.experimental.pallas import tpu as pltpu
from jax.experimental.pallas import tpu_sc as plsc

REQ_VEC = 400
LATENT = 64
LATENT_PAD = 128
BATCH = 1024

USER_BLOCK = 12800
ITEM_BLOCK = 5120


def _au_body(uT_ref, fsp_ref, au_ref):
    au_ref[...] = lax.dot_general(
        uT_ref[...], fsp_ref[...], (((0,), (0,)), ((), ())),
        preferred_element_type=jnp.float32,
    )


def _tc_all_users(uT, FSP):
    n_users = uT.shape[1]
    grid = (pl.cdiv(n_users, USER_BLOCK),)
    return pl.pallas_call(
        _au_body,
        grid=grid,
        in_specs=[
            pl.BlockSpec((REQ_VEC, USER_BLOCK), lambda i: (0, i)),
            pl.BlockSpec((REQ_VEC, LATENT_PAD), lambda i: (0, 0)),
        ],
        out_specs=pl.BlockSpec((USER_BLOCK, LATENT_PAD), lambda i: (i, 0)),
        out_shape=jax.ShapeDtypeStruct((n_users, LATENT_PAD), jnp.float32),
        compiler_params=pltpu.CompilerParams(vmem_limit_bytes=66060288),
    )(uT, FSP)


def _make_sc_gather(num_users):
    info = plsc.get_sparse_core_info()
    nw = info.num_cores * info.num_subcores
    assert BATCH % (8 * nw) == 0
    b_per_w = BATCH // nw
    mesh = plsc.VectorSubcoreMesh(core_axis_name="c", subcore_axis_name="s")

    @functools.partial(
        pl.kernel,
        mesh=mesh,
        out_type=jax.ShapeDtypeStruct((BATCH, LATENT_PAD), jnp.float32),
        scratch_types=[
            pltpu.VMEM((b_per_w,), jnp.int32),
            pltpu.VMEM((b_per_w, LATENT_PAD), jnp.float32),
            pltpu.SemaphoreType.DMA,
        ],
    )
    def gather_kernel(table_hbm, idx_hbm, out_hbm, idx_v, rows_v, sem):
        wid = lax.axis_index("s") * info.num_cores + lax.axis_index("c")
        base = wid * b_per_w
        pltpu.sync_copy(idx_hbm.at[pl.ds(base, b_per_w)], idx_v)
        pltpu.async_copy(table_hbm.at[idx_v], rows_v, sem).wait()
        pltpu.sync_copy(rows_v, out_hbm.at[pl.ds(base, b_per_w)])

    return gather_kernel


def _score_body(fst_ref, ug_ref, itemT_ref, out_ref):
    fiT = lax.dot_general(
        fst_ref[...], itemT_ref[...], (((1,), (0,)), ((), ())),
        preferred_element_type=jnp.float32,
    )
    logitsT = lax.dot_general(
        fiT, ug_ref[:, :LATENT], (((0,), (1,)), ((), ())),
        preferred_element_type=jnp.float32,
    )
    out_ref[...] = jax.nn.sigmoid(logitsT)


def _tc_score(FST, ug, itemT):
    n_items = itemT.shape[1]
    grid = (pl.cdiv(n_items, ITEM_BLOCK),)
    return pl.pallas_call(
        _score_body,
        grid=grid,
        in_specs=[
            pl.BlockSpec((LATENT, REQ_VEC), lambda i: (0, 0)),
            pl.BlockSpec((BATCH, LATENT_PAD), lambda i: (0, 0)),
            pl.BlockSpec((REQ_VEC, ITEM_BLOCK), lambda i: (0, i)),
        ],
        out_specs=pl.BlockSpec((ITEM_BLOCK, BATCH), lambda i: (i, 0)),
        out_shape=jax.ShapeDtypeStruct((n_items, BATCH), jnp.float32),
        compiler_params=pltpu.CompilerParams(vmem_limit_bytes=66060288),
    )(FST, ug, itemT)


@jax.jit
def kernel(users, user_vector, item_vector, FS):
    users = users.astype(jnp.int32)
    FSP = jnp.pad(FS, ((0, 0), (0, LATENT_PAD - LATENT)))
    au = _tc_all_users(user_vector.T, FSP)
    gather = _make_sc_gather(user_vector.shape[0])
    ug = gather(au, users)
    ratingT = _tc_score(FS.T, ug, item_vector.T)
    return ratingT.T
